# baseline (speedup 1.0000x reference)
import functools

import jax
import jax.numpy as jnp
from jax import lax
from jax.experimental import pallas as pl
from jax.experimental.pallas import tpu as pltpu
from jax.experimental.pallas import tpu_sc as plsc


def _make_kernel(batch, seq_len, dim, dtype):
    info = plsc.get_sparse_core_info()
    nc, ns = info.num_cores, info.num_subcores
    nw = nc * ns
    assert seq_len % nw == 0
    rows_per_w = seq_len // nw
    chunk = min(rows_per_w, 32)
    assert rows_per_w % chunk == 0
    n_chunks = rows_per_w // chunk

    mesh = plsc.VectorSubcoreMesh(core_axis_name="c", subcore_axis_name="s")

    @functools.partial(
        pl.kernel,
        mesh=mesh,
        out_type=jax.ShapeDtypeStruct((batch, seq_len, dim), dtype),
        scratch_types=[
            pltpu.VMEM((chunk, dim), dtype),
            pltpu.VMEM((chunk, dim), dtype),
            pltpu.SemaphoreType.DMA,
            pltpu.SemaphoreType.DMA,
            pltpu.SemaphoreType.DMA,
            pltpu.SemaphoreType.DMA,
        ],
    )
    def emb_broadcast(emb_hbm, out_hbm, buf0, buf1, rs0, rs1, ws0, ws1):
        wid = lax.axis_index("s") * nc + lax.axis_index("c")
        base = wid * rows_per_w
        bufs, rsems, wsems = (buf0, buf1), (rs0, rs1), (ws0, ws1)

        def read(i):
            return pltpu.async_copy(
                emb_hbm.at[pl.ds(base + i * chunk, chunk)], bufs[i % 2], rsems[i % 2]
            )

        pending_w = [[], []]
        reads = [None] * n_chunks
        reads[0] = read(0)
        for i in range(n_chunks):
            cur, nxt = i % 2, (i + 1) % 2
            if i + 1 < n_chunks:
                for d in pending_w[nxt]:
                    d.wait()
                pending_w[nxt] = []
                reads[i + 1] = read(i + 1)
            reads[i].wait()
            row0 = base + i * chunk
            for b in range(batch):
                pending_w[cur].append(
                    pltpu.async_copy(
                        bufs[cur], out_hbm.at[b, pl.ds(row0, chunk)], wsems[cur]
                    )
                )
        for lst in pending_w:
            for d in lst:
                d.wait()

    return emb_broadcast


def kernel(x, emb):
    batch, seq_len, _ = x.shape
    f = _make_kernel(batch, seq_len, emb.shape[1], emb.dtype)
    return f(emb)


# speedup vs baseline: 1.4483x; 1.4483x over previous
import functools

import jax
import jax.numpy as jnp
from jax.experimental import pallas as pl
from jax.experimental.pallas import tpu as pltpu


def _body(emb_ref, out_ref):
    out_ref[...] = jnp.broadcast_to(emb_ref[...][None], out_ref.shape)


def kernel(x, emb):
    batch, seq_len, dim = x.shape
    bs = 512
    grid = (seq_len // bs,)
    return pl.pallas_call(
        _body,
        grid=grid,
        in_specs=[pl.BlockSpec((bs, dim), lambda i: (i, 0))],
        out_specs=pl.BlockSpec((batch, bs, dim), lambda i: (0, i, 0)),
        out_shape=jax.ShapeDtypeStruct((batch, seq_len, dim), emb.dtype),
    )(emb)


# speedup vs baseline: 1.4786x; 1.0209x over previous
import functools

import jax
import jax.numpy as jnp
from jax.experimental import pallas as pl
from jax.experimental.pallas import tpu as pltpu


def _body(emb_ref, out_ref):
    out_ref[...] = jnp.broadcast_to(emb_ref[...][None], out_ref.shape)


def kernel(x, emb):
    batch, seq_len, dim = x.shape
    bs = 1024
    grid = (seq_len // bs,)
    return pl.pallas_call(
        _body,
        grid=grid,
        in_specs=[pl.BlockSpec((bs, dim), lambda i: (i, 0))],
        out_specs=pl.BlockSpec((batch, bs, dim), lambda i: (0, i, 0)),
        out_shape=jax.ShapeDtypeStruct((batch, seq_len, dim), emb.dtype),
    )(emb)
